# Initial kernel scaffold; baseline (speedup 1.0000x reference)
import jax
import jax.numpy as jnp
from jax.experimental import pallas as pl


def kernel(diagnosis_logits, labels, concept_scores):
    raise NotImplementedError("write your pallas kernel here")


# speedup vs baseline: 14.9642x; 14.9642x over previous
import jax
import jax.numpy as jnp
from jax.experimental import pallas as pl
from jax.experimental.pallas import tpu as pltpu

_ALPHA = 0.6
_BETA = 0.25
_GAMMA = 0.15
_TOPK = 10

_ROWS = 4096
_BLOCK_ROWS = 256
_GRID = _ROWS // _BLOCK_ROWS


def _body(dl_ref, lb_ref, cs_ref, bce_ref, sig_ref, tk_ref):
    step = pl.program_id(0)

    @pl.when(step == 0)
    def _init():
        bce_ref[0, 0] = 0.0
        sig_ref[0, 0] = 0.0
        tk_ref[0, 0] = 0.0

    x = dl_ref[...]
    y = lb_ref[...]
    per_elem = jnp.maximum(x, 0.0) - x * y + jnp.log1p(jnp.exp(-jnp.abs(x)))
    bce_ref[0, 0] += jnp.sum(per_elem)

    cs = cs_ref[...]
    probs = jax.nn.sigmoid(cs)
    sig_ref[0, 0] += jnp.sum(probs)

    work = cs
    rem = jnp.full((_BLOCK_ROWS, 1), float(_TOPK), dtype=jnp.float32)
    acc = jnp.zeros((_BLOCK_ROWS, 1), dtype=jnp.float32)
    for _ in range(_TOPK):
        m = jnp.max(work, axis=1, keepdims=True)
        eq = work == m
        c = jnp.sum(eq.astype(jnp.float32), axis=1, keepdims=True)
        take = jnp.minimum(c, rem)
        acc += take * jax.nn.sigmoid(m)
        rem -= take
        work = jnp.where(eq, -jnp.inf, work)
    tk_ref[0, 0] += jnp.sum(acc)


def kernel(diagnosis_logits, labels, concept_scores):
    n_dx = diagnosis_logits.size
    n_cs = concept_scores.size

    bce_sum, sig_sum, tk_sum = pl.pallas_call(
        _body,
        grid=(_GRID,),
        in_specs=[
            pl.BlockSpec((_BLOCK_ROWS, diagnosis_logits.shape[1]),
                         lambda i: (i, 0)),
            pl.BlockSpec((_BLOCK_ROWS, labels.shape[1]), lambda i: (i, 0)),
            pl.BlockSpec((_BLOCK_ROWS, concept_scores.shape[1]),
                         lambda i: (i, 0)),
        ],
        out_specs=[
            pl.BlockSpec(memory_space=pltpu.SMEM),
            pl.BlockSpec(memory_space=pltpu.SMEM),
            pl.BlockSpec(memory_space=pltpu.SMEM),
        ],
        out_shape=[
            jax.ShapeDtypeStruct((1, 1), jnp.float32),
            jax.ShapeDtypeStruct((1, 1), jnp.float32),
            jax.ShapeDtypeStruct((1, 1), jnp.float32),
        ],
    )(diagnosis_logits, labels, concept_scores)

    loss_dx = bce_sum[0, 0] / n_dx
    loss_concept_sparse = sig_sum[0, 0] / n_cs
    top_k_avg = tk_sum[0, 0] / (_ROWS * _TOPK)
    loss_concept_confidence = -top_k_avg
    total_loss = (_ALPHA * loss_dx + _BETA * loss_concept_sparse
                  + _GAMMA * loss_concept_confidence)
    return (total_loss, loss_dx, loss_concept_sparse,
            loss_concept_confidence, top_k_avg)
